# speedup vs baseline: 1.0320x; 1.0320x over previous
import functools

import jax
import jax.numpy as jnp
from jax import lax
from jax.experimental import pallas as pl
from jax.experimental.pallas import tpu as pltpu
from jax.experimental.pallas import tpu_sc as plsc

POS_OFFSET = 2
NC = 2
NS = 16
NW = NC * NS
LANES = 16
CH = 16


@functools.partial(jax.jit, static_argnums=(4, 5))
def _embed_lookup(ids3, pos3, embed_tokens, embed_positions, n_tokens, d):
    nch = n_tokens // (NW * CH)
    mesh = plsc.VectorSubcoreMesh(
        core_axis_name="c", subcore_axis_name="s",
        num_cores=NC, num_subcores=NS)

    @functools.partial(
        pl.kernel,
        out_type=jax.ShapeDtypeStruct((n_tokens, d), jnp.float32),
        mesh=mesh,
        scratch_types=[
            pltpu.VMEM((nch, CH), jnp.int32),
            pltpu.VMEM((nch, CH), jnp.int32),
            pltpu.VMEM((CH, d), jnp.float32),
            pltpu.VMEM((CH, d), jnp.float32),
            pltpu.SemaphoreType.DMA,
            pltpu.SemaphoreType.DMA,
        ],
    )
    def body(ids_hbm, pos_hbm, tok_tab, pos_tab, out_hbm,
             idx_t, idx_p, buf_t, buf_p, sem_t, sem_p):
        wid = lax.axis_index("s") * NC + lax.axis_index("c")
        pltpu.sync_copy(ids_hbm.at[wid], idx_t)
        pltpu.sync_copy(pos_hbm.at[wid], idx_p)
        for j in range(nch):
            idx_p[j] = idx_p[j] + POS_OFFSET
        base = wid * (nch * CH)
        for j in range(nch):
            ct = pltpu.async_copy(tok_tab.at[idx_t.at[j]], buf_t, sem_t)
            cp = pltpu.async_copy(pos_tab.at[idx_p.at[j]], buf_p, sem_p)
            ct.wait()
            cp.wait()

            @pl.loop(0, d // LANES)
            def _(i):
                off = pl.multiple_of(i * LANES, LANES)
                for r in range(CH):
                    buf_t[r, pl.ds(off, LANES)] = (
                        buf_t[r, pl.ds(off, LANES)]
                        + buf_p[r, pl.ds(off, LANES)])

            pltpu.sync_copy(buf_t, out_hbm.at[pl.ds(base + j * CH, CH)])

    return body(ids3, pos3, embed_tokens, embed_positions)


def kernel(input_ids, positions, embed_tokens, embed_positions):
    b, s = input_ids.shape
    d = embed_tokens.shape[1]
    n = b * s
    nch = n // (NW * CH)
    ids3 = input_ids.reshape(NW, nch, CH).astype(jnp.int32)
    pos3 = positions.reshape(NW, nch, CH).astype(jnp.int32)
    out = _embed_lookup(ids3, pos3, embed_tokens, embed_positions, n, d)
    return out.reshape(b, s, d)


# speedup vs baseline: 1.6171x; 1.5669x over previous
import functools

import jax
import jax.numpy as jnp
from jax import lax
from jax.experimental import pallas as pl
from jax.experimental.pallas import tpu as pltpu
from jax.experimental.pallas import tpu_sc as plsc

POS_OFFSET = 2
NC = 2
NS = 16
NW = NC * NS
LANES = 16
CH = 8
NSLOT = 2


@functools.partial(jax.jit, static_argnums=(4, 5))
def _embed_lookup(ids3, pos3, embed_tokens, embed_positions, n_tokens, d):
    rpw = n_tokens // NW
    nch = rpw // CH
    mesh = plsc.VectorSubcoreMesh(
        core_axis_name="c", subcore_axis_name="s",
        num_cores=NC, num_subcores=NS)

    @functools.partial(
        pl.kernel,
        out_type=jax.ShapeDtypeStruct((n_tokens, d), jnp.float32),
        mesh=mesh,
        scratch_types=[
            pltpu.VMEM((rpw,), jnp.int32),
            pltpu.VMEM((rpw,), jnp.int32),
            pltpu.VMEM((NSLOT, CH, d), jnp.float32),
            pltpu.VMEM((NSLOT, CH, d), jnp.float32),
            pltpu.VMEM((NSLOT, CH, d), jnp.float32),
            [pltpu.SemaphoreType.DMA] * NSLOT,
            [pltpu.SemaphoreType.DMA] * NSLOT,
            [pltpu.SemaphoreType.DMA] * NSLOT,
        ],
    )
    def body(ids_hbm, pos_hbm, tok_tab, pos_tab, out_hbm,
             idx_t, idx_p, buf_t, buf_p, buf_o, sems_t, sems_p, sems_o):
        wid = lax.axis_index("s") * NC + lax.axis_index("c")
        pltpu.sync_copy(ids_hbm.at[wid], idx_t)
        pltpu.sync_copy(pos_hbm.at[wid], idx_p)
        for j in range(rpw // LANES):
            sl = pl.ds(j * LANES, LANES)
            idx_p[sl] = idx_p[sl] + POS_OFFSET
        base = wid * rpw

        def issue_gather(j):
            slot = j % NSLOT
            ct = pltpu.async_copy(
                tok_tab.at[idx_t.at[pl.ds(j * CH, CH)]],
                buf_t.at[slot], sems_t[slot])
            cp = pltpu.async_copy(
                pos_tab.at[idx_p.at[pl.ds(j * CH, CH)]],
                buf_p.at[slot], sems_p[slot])
            return ct, cp

        pending_g = {}
        pending_o = {}
        for j in range(min(NSLOT, nch)):
            pending_g[j] = issue_gather(j)

        for j in range(nch):
            slot = j % NSLOT
            ct, cp = pending_g.pop(j)
            ct.wait()
            cp.wait()
            if j - NSLOT in pending_o:
                pending_o.pop(j - NSLOT).wait()

            @pl.loop(0, d // LANES)
            def _(i):
                off = pl.multiple_of(i * LANES, LANES)
                for r in range(CH):
                    buf_o[slot, r, pl.ds(off, LANES)] = (
                        buf_t[slot, r, pl.ds(off, LANES)]
                        + buf_p[slot, r, pl.ds(off, LANES)])

            pending_o[j] = pltpu.async_copy(
                buf_o.at[slot], out_hbm.at[pl.ds(base + j * CH, CH)],
                sems_o[slot])
            if j + NSLOT < nch:
                pending_g[j + NSLOT] = issue_gather(j + NSLOT)

        for j in sorted(pending_o):
            pending_o.pop(j).wait()

    return body(ids3, pos3, embed_tokens, embed_positions)


def kernel(input_ids, positions, embed_tokens, embed_positions):
    b, s = input_ids.shape
    d = embed_tokens.shape[1]
    n = b * s
    ids3 = input_ids.reshape(NW, n // NW).astype(jnp.int32)
    pos3 = positions.reshape(NW, n // NW).astype(jnp.int32)
    out = _embed_lookup(ids3, pos3, embed_tokens, embed_positions, n, d)
    return out.reshape(b, s, d)
